# baseline (speedup 1.0000x reference)
import jax
import jax.numpy as jnp
from jax.experimental import pallas as pl
from jax.experimental.pallas import tpu as pltpu

INPUT_DIM = 128
TIME_STEP = 32
HIDDEN = 256
N_NODES = 100
N_CAT = 5
N_PER = 20

E_IN = 1792
E_OUT = 32
NCPAD = 8

_TR = (((1,), (1,)), ((), ()))
_CC = (((0,), (0,)), ((), ()))


def _dot_t(x, w):
    return jax.lax.dot_general(x, w, _TR, preferred_element_type=jnp.float32)


def _col(row, n):
    return jnp.concatenate([row[:, i:i + 1] for i in range(n)], axis=0)


def _gru_block(seq_ref, wih_ref, whh_ref, bih_ref, bhh_ref, h_scr, h2_scr):
    H = HIDDEN
    bf16 = jnp.bfloat16
    wih = wih_ref[...].astype(bf16)
    whh = whh_ref[...].astype(bf16)
    b_sum = bih_ref[...] + bhh_ref[...]
    bih_n = bih_ref[:, 2 * H:]
    bhh_n = bhh_ref[:, 2 * H:]
    h = jnp.zeros((N_NODES, H), jnp.float32)
    for t in range(TIME_STEP):
        x_t = seq_ref[:, t * INPUT_DIM:(t + 1) * INPUT_DIM].astype(bf16)
        gi = jax.lax.dot_general(x_t, wih, _TR,
                                 preferred_element_type=jnp.float32)
        gh = jax.lax.dot_general(h.astype(bf16), whh, _TR,
                                 preferred_element_type=jnp.float32)
        g = gi + gh + b_sum
        r = jax.nn.sigmoid(g[:, 0:H])
        z = jax.nn.sigmoid(g[:, H:2 * H])
        n = jnp.tanh(gi[:, 2 * H:] + bih_n + r * (gh[:, 2 * H:] + bhh_n))
        h = (1.0 - z) * n + z * h
        h_scr[...] = h
        for b in range(N_NODES):
            h2_scr[t:t + 1, b * H:(b + 1) * H] = h_scr[b:b + 1, :]


def _gat(xp_feat, asrc_col, adst_col, edge, n_loop, n_pad, n_edge):
    n_real = edge.shape[1]
    loop_row = jax.lax.broadcasted_iota(jnp.int32, (1, n_loop), 1)
    pad_row = jnp.full((1, n_edge - n_real - n_loop), -1, jnp.int32)
    src = jnp.concatenate([edge[0:1, :], loop_row, pad_row], axis=1)
    dst = jnp.concatenate([edge[1:2, :], loop_row, pad_row], axis=1)
    node_iota = jax.lax.broadcasted_iota(jnp.int32, (n_pad, n_edge), 0)
    oh_src = (src == node_iota).astype(jnp.float32)
    oh_dst = (dst == node_iota).astype(jnp.float32)
    asrc_e = jax.lax.dot_general(asrc_col, oh_src, _CC,
                                 preferred_element_type=jnp.float32)
    adst_e = jax.lax.dot_general(adst_col, oh_dst, _CC,
                                 preferred_element_type=jnp.float32)
    pre = asrc_e + adst_e
    alpha = jnp.where(pre >= 0, pre, 0.2 * pre)
    masked = jnp.where(oh_dst > 0, alpha, -1e30)
    m_col = jnp.max(masked, axis=1, keepdims=True)
    m_e = jax.lax.dot_general(m_col, oh_dst, _CC,
                              preferred_element_type=jnp.float32)
    e = jnp.exp(alpha - m_e)
    s_col = jnp.sum(oh_dst * e, axis=1, keepdims=True)
    s_e = jax.lax.dot_general(s_col, oh_dst, _CC,
                              preferred_element_type=jnp.float32)
    a_e = e / (s_e + 1e-16)
    xp_src = jax.lax.dot_general(oh_src, xp_feat, _CC,
                                 preferred_element_type=jnp.float32)
    return jnp.dot(oh_dst * a_e, xp_src,
                   preferred_element_type=jnp.float32)


def _net_kernel(seq_ref, wih_ref, whh_ref, bih_ref, bhh_ref,
                watt_ref, ie_ref, oe_ref, wpool_ref,
                wgin_ref, bgin_ref, wgcat_ref, bgcat_ref,
                wf_ref, bf_ref, pr_ref,
                reg_ref, cls_ref, h_scr, h2_scr):
    H = HIDDEN
    f32 = jnp.float32
    pr = pr_ref[...]

    _gru_block(seq_ref, wih_ref, whh_ref, bih_ref, bhh_ref, h_scr, h2_scr)

    h2 = h2_scr[...]
    aw = (jnp.dot(watt_ref[...], h2, preferred_element_type=f32)
          + _col(pr[6:7, :], TIME_STEP))
    m = jnp.max(aw, axis=0, keepdims=True)
    e = jnp.exp(aw - m)
    ap = e / jnp.sum(e, axis=0, keepdims=True)
    att = jnp.sum(ap * h2, axis=0, keepdims=True)
    wav = jnp.concatenate(
        [att[:, b * H:(b + 1) * H] for b in range(N_NODES)], axis=0)

    xp_in = _dot_t(wav, wgin_ref[...])
    asrc_in = jnp.sum(xp_in * pr[0:1, :], axis=1, keepdims=True)
    adst_in = jnp.sum(xp_in * pr[1:2, :], axis=1, keepdims=True)
    inner = _gat(xp_in, asrc_in, adst_in, ie_ref[...], N_NODES, N_NODES,
                 E_IN)
    inner = inner + bgin_ref[...]

    blocks = []
    for c in range(N_CAT):
        sel = (jax.lax.broadcasted_iota(jnp.int32, (N_PER, N_NODES), 1)
               == c * N_PER
               + jax.lax.broadcasted_iota(jnp.int32, (N_PER, N_NODES), 0)
               ).astype(f32)
        blocks.append(jnp.dot(sel, wav, preferred_element_type=f32))
    pool_in = jnp.concatenate(blocks, axis=1)
    awp = (jnp.dot(wpool_ref[...], pool_in, preferred_element_type=f32)
           + _col(pr[7:8, :], N_PER))
    mp = jnp.max(awp, axis=0, keepdims=True)
    ep = jnp.exp(awp - mp)
    app = ep / jnp.sum(ep, axis=0, keepdims=True)
    catv = jnp.sum(app * pool_in, axis=0, keepdims=True)
    cat_rows = jnp.concatenate(
        [catv[:, c * H:(c + 1) * H] for c in range(N_CAT)]
        + [jnp.zeros((NCPAD - N_CAT, H), f32)], axis=0)

    xp_cat = _dot_t(cat_rows, wgcat_ref[...])
    asrc_cat = jnp.sum(xp_cat * pr[2:3, :], axis=1, keepdims=True)
    adst_cat = jnp.sum(xp_cat * pr[3:4, :], axis=1, keepdims=True)
    catg = _gat(xp_cat, asrc_cat, adst_cat, oe_ref[...], N_CAT, NCPAD,
                E_OUT)
    catg = catg + bgcat_ref[...]

    row = jax.lax.broadcasted_iota(jnp.int32, (N_NODES, NCPAD), 0) // N_PER
    col = jax.lax.broadcasted_iota(jnp.int32, (N_NODES, NCPAD), 1)
    assign = (row == col).astype(f32)
    cat_exp = jnp.dot(assign, catg, preferred_element_type=f32)
    wf = wf_ref[...]
    fusion = (
        _dot_t(wav, wf[:, 0:H])
        + _dot_t(cat_exp, wf[:, H:2 * H])
        + _dot_t(inner, wf[:, 2 * H:])
        + bf_ref[...]
    )
    fusion = jnp.maximum(fusion, 0.0)
    reg_ref[...] = (
        jnp.sum(fusion * pr[4:5, :], axis=1, keepdims=True)
        + pr[7:8, N_PER:N_PER + 1]
    )
    cls_ref[...] = jax.nn.sigmoid(
        jnp.sum(fusion * pr[5:6, :], axis=1, keepdims=True)
        + pr[7:8, N_PER + 1:N_PER + 2]
    )


@jax.jit
def kernel(weekly_batch, inner_edge, outer_edge, W_ih, W_hh, b_ih, b_hh,
           W_att_enc, b_att_enc, W_att_pool, b_att_pool, W_gat_in, a_src_in,
           a_dst_in, b_gat_in, W_gat_cat, a_src_cat, a_dst_cat, b_gat_cat,
           W_f, b_f, W_r, b_r, W_c, b_c):
    f32 = jnp.float32
    H = HIDDEN

    row6 = jnp.concatenate([b_att_enc, jnp.zeros((H - TIME_STEP,), f32)])
    row7 = jnp.concatenate(
        [b_att_pool, b_r, b_c, jnp.zeros((H - N_PER - 2,), f32)])
    params = jnp.stack(
        [a_src_in, a_dst_in, a_src_cat, a_dst_cat,
         W_r.reshape(-1), W_c.reshape(-1), row6, row7], axis=0)

    seq_flat = weekly_batch.reshape(N_NODES, TIME_STEP * INPUT_DIM)
    reg, cls = pl.pallas_call(
        _net_kernel,
        out_shape=(
            jax.ShapeDtypeStruct((N_NODES, 1), f32),
            jax.ShapeDtypeStruct((N_NODES, 1), f32),
        ),
        scratch_shapes=[
            pltpu.VMEM((N_NODES, H), f32),
            pltpu.VMEM((TIME_STEP, N_NODES * H), f32),
        ],
    )(
        seq_flat, W_ih, W_hh, b_ih.reshape(1, -1), b_hh.reshape(1, -1),
        W_att_enc, inner_edge, outer_edge, W_att_pool,
        W_gat_in, b_gat_in.reshape(1, -1),
        W_gat_cat, b_gat_cat.reshape(1, -1),
        W_f, b_f.reshape(1, -1), params,
    )
    return reg.reshape(-1), cls.reshape(-1)
